# baseline (speedup 1.0000x reference)
import jax
import jax.numpy as jnp
from jax.experimental import pallas as pl
from jax.experimental.pallas import tpu as pltpu

DICT_SIZE = 1000
_NBUF = 8
_BBLK = 512


def _onehot_ring_kernel(NCH, inp_ref, out_ref, scratch, sems):
    i = pl.program_id(0)
    slot = jax.lax.rem(i, _NBUF)
    s = i // (1024 // _BBLK)
    c = jax.lax.rem(i, 1024 // _BBLK)

    def _copy(step, sl):
        st = step // (1024 // _BBLK)
        ct = jax.lax.rem(step, 1024 // _BBLK)
        return pltpu.make_async_copy(
            scratch.at[pl.ds(sl, 1)],
            out_ref.at[pl.ds(st, 1), pl.ds(ct * _BBLK, _BBLK)],
            sems.at[sl],
        )

    @pl.when(i >= _NBUF)
    def _():
        _copy(i - _NBUF, slot).wait()

    idx = inp_ref[0, 0, pl.ds(c * _BBLK, _BBLK)]
    d = jax.lax.broadcasted_iota(jnp.int32, (_BBLK, DICT_SIZE), 1)
    scratch[pl.ds(slot, 1)] = (d == idx[:, None]).astype(jnp.float32)[None]
    _copy(i, slot).start()

    @pl.when(i == NCH - 1)
    def _():
        for k in range(_NBUF):
            bi = NCH - _NBUF + k
            _copy(bi, bi % _NBUF).wait()


def kernel(input, teacher_forcing):
    S, B = input.shape
    inp3 = input.reshape(S, 1, B)
    nch = S * (B // _BBLK)
    p2 = pl.pallas_call(
        lambda *refs: _onehot_ring_kernel(nch, *refs),
        grid=(nch,),
        in_specs=[
            pl.BlockSpec((1, 1, B), lambda i: (i // (1024 // _BBLK), 0, 0))
        ],
        out_specs=pl.BlockSpec(memory_space=pl.ANY),
        out_shape=jax.ShapeDtypeStruct((S, B, DICT_SIZE), jnp.float32),
        scratch_shapes=[
            pltpu.VMEM((_NBUF, _BBLK, DICT_SIZE), jnp.float32),
            pltpu.SemaphoreType.DMA((_NBUF,)),
        ],
        compiler_params=pltpu.CompilerParams(
            dimension_semantics=("arbitrary",),
        ),
    )(inp3)
    return (p2, input)


# speedup vs baseline: 1.9158x; 1.9158x over previous
import functools

import jax
import jax.numpy as jnp
from jax import lax
from jax.experimental import pallas as pl
from jax.experimental.pallas import tpu as pltpu
from jax.experimental.pallas import tpu_sc as plsc

DICT_SIZE = 1000
_NC = 2
_NS = 16
_BAND = 200
_CH = 40
_NB = DICT_SIZE // _BAND


def _sc_onehot_body(S, B, idx_hbm, out_hbm, idx_v, buf, sem):
    nw = _NC * _NS
    wid = lax.axis_index("s") * _NC + lax.axis_index("c")
    nunits = S * _NB

    zeros16 = jnp.zeros((16,), jnp.float32)
    ones16 = jnp.ones((16,), jnp.float32)
    lane = lax.iota(jnp.int32, 16)
    nbch = B // 16

    def _zero(i, carry):
        r = i // nbch
        c = i - r * nbch
        buf[r, pl.ds(c * 16, 16)] = zeros16
        return carry

    lax.fori_loop(0, _CH * nbch, _zero, 0, unroll=8)

    def _scan(d0, val16):
        def _bchunk(ck, carry):
            idxs = idx_v[pl.ds(ck * 16, 16)]
            dloc = idxs - d0
            m = jnp.logical_and(dloc >= 0, dloc < _CH)
            plsc.store_scatter(buf, [dloc, lane + ck * 16], val16, mask=m)
            return carry

        lax.fori_loop(0, nbch, _bchunk, 0, unroll=4)

    def _unit(k, carry):
        u = wid + k * nw

        @pl.when(u < nunits)
        def _():
            s = u // _NB
            part = u - s * _NB
            pltpu.sync_copy(idx_hbm.at[s], idx_v)
            for c in range(_BAND // _CH):
                d0 = part * _BAND + c * _CH
                _scan(d0, ones16)
                pltpu.sync_copy(buf, out_hbm.at[s, pl.ds(d0, _CH), :])
                _scan(d0, zeros16)

        return carry

    lax.fori_loop(0, (nunits + nw - 1) // nw, _unit, 0)


def kernel(input, teacher_forcing):
    S, B = input.shape
    idx = input.astype(jnp.int32)

    sc_call = pl.kernel(
        functools.partial(_sc_onehot_body, S, B),
        out_type=jax.ShapeDtypeStruct((S, DICT_SIZE, B), jnp.float32),
        mesh=plsc.VectorSubcoreMesh(core_axis_name="c", subcore_axis_name="s"),
        scratch_types=[
            pltpu.VMEM((B,), jnp.int32),
            pltpu.VMEM((_CH, B), jnp.float32),
            pltpu.SemaphoreType.DMA,
        ],
        compiler_params=pltpu.CompilerParams(needs_layout_passes=False),
    )
    p2 = jnp.swapaxes(sc_call(idx), 1, 2)
    return (p2, input)


# speedup vs baseline: 2.0586x; 1.0745x over previous
import functools

import jax
import jax.numpy as jnp
from jax import lax
from jax.experimental import pallas as pl
from jax.experimental.pallas import tpu as pltpu
from jax.experimental.pallas import tpu_sc as plsc

DICT_SIZE = 1000
_NC = 2
_NS = 16
_CH = 40


def _sc_onehot_body(S, B, idx_hbm, out_hbm, idx0, idx1, buf0, buf1, sem0, sem1):
    nw = _NC * _NS
    wid = lax.axis_index("s") * _NC + lax.axis_index("c")
    nbands = DICT_SIZE // _CH
    nunits = S * nbands
    nk = (nunits + nw - 1) // nw
    idxs_v = (idx0, idx1)
    bufs = (buf0, buf1)
    sems = (sem0, sem1)

    zeros16 = jnp.zeros((16,), jnp.float32)
    ones16 = jnp.ones((16,), jnp.float32)
    lane = lax.iota(jnp.int32, 16)
    nbch = B // 16

    def _zero(i, carry):
        r = i // nbch
        c = i - r * nbch
        buf0[r, pl.ds(c * 16, 16)] = zeros16
        buf1[r, pl.ds(c * 16, 16)] = zeros16
        return carry

    lax.fori_loop(0, _CH * nbch, _zero, 0, unroll=8)

    def _sd(k):
        u = wid + k * nw
        s = u // nbands
        d0 = (u - s * nbands) * _CH
        return u, s, d0

    def _scan(slot, d0, val16):
        def _bchunk(ck, carry):
            idxs = idxs_v[slot][pl.ds(ck * 16, 16)]
            dloc = idxs - d0
            m = jnp.logical_and(dloc >= 0, dloc < _CH)
            plsc.store_scatter(bufs[slot], [dloc, lane + ck * 16], val16, mask=m)
            return carry

        lax.fori_loop(0, nbch, _bchunk, 0, unroll=4)

    def _copy(slot, s, d0):
        return pltpu.make_async_copy(
            bufs[slot], out_hbm.at[s, pl.ds(d0, _CH), :], sems[slot]
        )

    def _pair(g, carry):
        for b in range(2):
            k = 2 * g + b
            u, s, d0 = _sd(k)

            @pl.when(g >= 1)
            def _():
                u2, s2, d02 = _sd(k - 2)
                _copy(b, s2, d02).wait()
                _scan(b, d02, zeros16)

            @pl.when(u < nunits)
            def _():
                pltpu.sync_copy(idx_hbm.at[s], idxs_v[b])
                _scan(b, d0, ones16)
                _copy(b, s, d0).start()

        return carry

    lax.fori_loop(0, nk // 2, _pair, 0)

    u, s, d0 = _sd(nk - 2)
    _copy(0, s, d0).wait()
    u, s, d0 = _sd(nk - 1)

    @pl.when(u < nunits)
    def _():
        _copy(1, s, d0).wait()


def kernel(input, teacher_forcing):
    S, B = input.shape
    idx = input.astype(jnp.int32)

    sc_call = pl.kernel(
        functools.partial(_sc_onehot_body, S, B),
        out_type=jax.ShapeDtypeStruct((S, DICT_SIZE, B), jnp.float32),
        mesh=plsc.VectorSubcoreMesh(core_axis_name="c", subcore_axis_name="s"),
        scratch_types=[
            pltpu.VMEM((B,), jnp.int32),
            pltpu.VMEM((B,), jnp.int32),
            pltpu.VMEM((_CH, B), jnp.float32),
            pltpu.VMEM((_CH, B), jnp.float32),
            pltpu.SemaphoreType.DMA,
            pltpu.SemaphoreType.DMA,
        ],
        compiler_params=pltpu.CompilerParams(needs_layout_passes=False),
    )
    p2 = jnp.swapaxes(sc_call(idx), 1, 2)
    return (p2, input)


# speedup vs baseline: 2.8000x; 1.3601x over previous
import functools

import jax
import jax.numpy as jnp
from jax import lax
from jax.experimental import pallas as pl
from jax.experimental.pallas import tpu as pltpu
from jax.experimental.pallas import tpu_sc as plsc

DICT_SIZE = 1000
_NC = 2
_NS = 16
_CH = 40


def _sc_onehot_body(S, B, idx_hbm, out_hbm, idx0, idx1, buf0, buf1, sem0, sem1):
    nw = _NC * _NS
    wid = lax.axis_index("s") * _NC + lax.axis_index("c")
    nbands = DICT_SIZE // _CH
    nunits = S * nbands
    nk = nunits // nw + 1
    rem = nunits - nw * (nk - 1)
    start = wid * (nk - 1) + jnp.minimum(wid, rem)
    nu = (nk - 1) + (wid < rem).astype(jnp.int32)
    idxs_v = (idx0, idx1)
    bufs = (buf0, buf1)
    sems = (sem0, sem1)

    zeros16 = jnp.zeros((16,), jnp.float32)
    ones16 = jnp.ones((16,), jnp.float32)
    lane = lax.iota(jnp.int32, 16)
    nbch = B // 16

    def _zero(i, carry):
        r = i // nbch
        c = i - r * nbch
        buf0[r, pl.ds(c * 16, 16)] = zeros16
        buf1[r, pl.ds(c * 16, 16)] = zeros16
        return carry

    lax.fori_loop(0, _CH * nbch, _zero, 0, unroll=8)

    def _sd(k):
        u = start + k
        s = u // nbands
        d0 = (u - s * nbands) * _CH
        return u, s, d0

    def _scan(slot, d0, val16):
        def _bchunk(ck, carry):
            idxs = idxs_v[slot][pl.ds(ck * 16, 16)]
            dloc = idxs - d0
            m = plsc.bitcast(dloc, jnp.uint32) < jnp.uint32(_CH)
            plsc.store_scatter(bufs[slot], [dloc, lane + ck * 16], val16, mask=m)
            return carry

        lax.fori_loop(0, nbch, _bchunk, 0, unroll=8)

    def _copy(slot, s, d0):
        return pltpu.make_async_copy(
            bufs[slot], out_hbm.at[s, pl.ds(d0, _CH), :], sems[slot]
        )

    def _pair(g, carry):
        for b in range(2):
            k = 2 * g + b
            u, s, d0 = _sd(k)
            u2, s2, d02 = _sd(k - 2)

            @pl.when(g >= 1)
            def _():
                _copy(b, s2, d02).wait()
                _scan(b, d02, zeros16)

            @pl.when(k < nu)
            def _():
                @pl.when(jnp.logical_or(k <= 1, s != s2))
                def _():
                    pltpu.sync_copy(idx_hbm.at[s], idxs_v[b])

                _scan(b, d0, ones16)
                _copy(b, s, d0).start()

        return carry

    lax.fori_loop(0, nk // 2, _pair, 0)

    u, s, d0 = _sd(nk - 2)
    _copy(0, s, d0).wait()
    u, s, d0 = _sd(nk - 1)

    @pl.when(nk - 1 < nu)
    def _():
        _copy(1, s, d0).wait()


def kernel(input, teacher_forcing):
    S, B = input.shape
    idx = input.astype(jnp.int32)

    sc_call = pl.kernel(
        functools.partial(_sc_onehot_body, S, B),
        out_type=jax.ShapeDtypeStruct((S, DICT_SIZE, B), jnp.float32),
        mesh=plsc.VectorSubcoreMesh(core_axis_name="c", subcore_axis_name="s"),
        scratch_types=[
            pltpu.VMEM((B,), jnp.int32),
            pltpu.VMEM((B,), jnp.int32),
            pltpu.VMEM((_CH, B), jnp.float32),
            pltpu.VMEM((_CH, B), jnp.float32),
            pltpu.SemaphoreType.DMA,
            pltpu.SemaphoreType.DMA,
        ],
        compiler_params=pltpu.CompilerParams(needs_layout_passes=False),
    )
    p2 = jnp.swapaxes(sc_call(idx), 1, 2)
    return (p2, input)
